# Initial kernel scaffold; baseline (speedup 1.0000x reference)
import jax
import jax.numpy as jnp
from jax.experimental import pallas as pl


def kernel(u_emb, i_emb, edge_index, rui, riu, W1, W2, pV, pK):
    raise NotImplementedError("write your pallas kernel here")


# speedup vs baseline: 2.3031x; 2.3031x over previous
import functools

import jax
import jax.numpy as jnp
import numpy as np
from jax import lax
from jax.experimental import pallas as pl
from jax.experimental.pallas import tpu as pltpu
from jax.experimental.pallas import tpu_sc as plsc

NC = 2
NS = 16
NW = NC * NS
LB = 128
EB = 512


def _messages_tc(u_emb, i_emb, W1, W2):
    n, d = u_emb.shape
    blk = 512

    def body(u_ref, i_ref, w1_ref, w2_ref, um_ref, im_ref):
        dn = (((1,), (1,)), ((), ()))
        um_ref[...] = lax.dot_general(u_ref[...], w2_ref[...], dn,
                                      preferred_element_type=jnp.float32)
        im_ref[...] = lax.dot_general(i_ref[...], w1_ref[...], dn,
                                      preferred_element_type=jnp.float32)

    return pl.pallas_call(
        body,
        grid=(n // blk,),
        in_specs=[
            pl.BlockSpec((blk, d), lambda i: (i, 0)),
            pl.BlockSpec((blk, d), lambda i: (i, 0)),
            pl.BlockSpec((d, d), lambda i: (0, 0)),
            pl.BlockSpec((d, d), lambda i: (0, 0)),
        ],
        out_specs=[
            pl.BlockSpec((blk, d), lambda i: (i, 0)),
            pl.BlockSpec((blk, d), lambda i: (i, 0)),
        ],
        out_shape=[
            jax.ShapeDtypeStruct((n, d), jnp.float32),
            jax.ShapeDtypeStruct((n, d), jnp.float32),
        ],
    )(u_emb, i_emb, W1, W2)


def _tables_tc(um, im, pvp, pkp):
    n, d = um.shape
    p = pvp.shape[0]
    blk = 512

    def body(um_ref, im_ref, imf_ref, pv_ref, pk_ref, e_ref, a_ref, b_ref):
        dn = (((1,), (1,)), ((), ()))
        e_ref[...] = lax.dot_general(um_ref[...], imf_ref[...], dn,
                                     preferred_element_type=jnp.float32)
        a_ref[...] = lax.dot_general(um_ref[...], pv_ref[...], dn,
                                     preferred_element_type=jnp.float32)
        b_ref[...] = lax.dot_general(im_ref[...], pk_ref[...], dn,
                                     preferred_element_type=jnp.float32)

    return pl.pallas_call(
        body,
        grid=(n // blk,),
        in_specs=[
            pl.BlockSpec((blk, d), lambda i: (i, 0)),
            pl.BlockSpec((blk, d), lambda i: (i, 0)),
            pl.BlockSpec((n, d), lambda i: (0, 0)),
            pl.BlockSpec((p, d), lambda i: (0, 0)),
            pl.BlockSpec((p, d), lambda i: (0, 0)),
        ],
        out_specs=[
            pl.BlockSpec((blk, n), lambda i: (i, 0)),
            pl.BlockSpec((blk, p), lambda i: (i, 0)),
            pl.BlockSpec((blk, p), lambda i: (i, 0)),
        ],
        out_shape=[
            jax.ShapeDtypeStruct((n, n), jnp.float32),
            jax.ShapeDtypeStruct((n, p), jnp.float32),
            jax.ShapeDtypeStruct((n, p), jnp.float32),
        ],
    )(um, im, im, pvp, pkp)


def _edge_exp_sc(e_flat, a_flat, b_flat, rowe, rowa, rowb,
                 lane_e, lane_a, lane_b, inv_sqrt_d):
    E = rowe.shape[0]
    epw = E // NW
    nb = epw // LB
    mesh = plsc.VectorSubcoreMesh(core_axis_name="c", subcore_axis_name="s")

    @functools.partial(
        pl.kernel,
        out_type=[
            jax.ShapeDtypeStruct((E,), jnp.float32),
            jax.ShapeDtypeStruct((E,), jnp.float32),
        ],
        mesh=mesh,
        compiler_params=pltpu.CompilerParams(needs_layout_passes=False),
        scratch_types=[
            pltpu.VMEM((LB,), jnp.int32),
            pltpu.VMEM((LB,), jnp.int32),
            pltpu.VMEM((LB,), jnp.int32),
            pltpu.VMEM((LB,), jnp.int32),
            pltpu.VMEM((LB,), jnp.int32),
            pltpu.VMEM((LB,), jnp.int32),
            pltpu.VMEM((LB, 128), jnp.float32),
            pltpu.VMEM((LB, 128), jnp.float32),
            pltpu.VMEM((LB, 128), jnp.float32),
            pltpu.VMEM((LB,), jnp.float32),
            pltpu.VMEM((LB,), jnp.float32),
        ],
    )
    def kern(ef_h, af_h, bf_h, re_h, ra_h, rb_h, le_h, la_h, lb_h,
             ea_h, eb_h,
             ire, ira, irb, ile, ila, ilb, er, ar, br, ea_v, eb_v):
        cid = lax.axis_index("c")
        sid = lax.axis_index("s")
        wid = cid * NS + sid
        iota16 = lax.iota(jnp.int32, 16)

        def block(b, _):
            base = wid * epw + b * LB
            pltpu.sync_copy(re_h.at[pl.ds(base, LB)], ire)
            pltpu.sync_copy(ra_h.at[pl.ds(base, LB)], ira)
            pltpu.sync_copy(rb_h.at[pl.ds(base, LB)], irb)
            pltpu.sync_copy(le_h.at[pl.ds(base, LB)], ile)
            pltpu.sync_copy(la_h.at[pl.ds(base, LB)], ila)
            pltpu.sync_copy(lb_h.at[pl.ds(base, LB)], ilb)
            pltpu.sync_copy(ef_h.at[ire], er)
            pltpu.sync_copy(af_h.at[ira], ar)
            pltpu.sync_copy(bf_h.at[irb], br)

            def group(g, _):
                rows16 = iota16 + g * 16
                le16 = ile[pl.ds(g * 16, 16)]
                la16 = ila[pl.ds(g * 16, 16)]
                lb16 = ilb[pl.ds(g * 16, 16)]
                ev = plsc.load_gather(er, [rows16, le16])
                av = plsc.load_gather(ar, [rows16, la16])
                bv = plsc.load_gather(br, [rows16, lb16])
                ea_v[pl.ds(g * 16, 16)] = jnp.exp((ev + av) * inv_sqrt_d)
                eb_v[pl.ds(g * 16, 16)] = jnp.exp((ev + bv) * inv_sqrt_d)
                return 0
            lax.fori_loop(0, LB // 16, group, 0)

            pltpu.sync_copy(ea_v, ea_h.at[pl.ds(base, LB)])
            pltpu.sync_copy(eb_v, eb_h.at[pl.ds(base, LB)])
            return 0
        lax.fori_loop(0, nb, block, 0)

    return kern(e_flat, a_flat, b_flat, rowe, rowa, rowb,
                lane_e, lane_a, lane_b)


def _aggregate_tc(seg, msg_idx_t, pos_idx_t, w, msg_tab, pos_tab):
    E = seg.shape[1]
    n, d = msg_tab.shape
    p = pos_tab.shape[0]
    nblk = E // EB

    def body(seg_ref, mi_ref, pi_ref, w_ref, mt_ref, pt_ref, out_ref, acc):
        e = pl.program_id(0)

        @pl.when(e == 0)
        def _():
            acc[...] = jnp.zeros_like(acc)

        io_l = lax.broadcasted_iota(jnp.int32, mi_ref.shape, 1)
        mi = jnp.sum(jnp.where(io_l == e, mi_ref[...], 0), axis=1,
                     keepdims=True)
        pi = jnp.sum(jnp.where(io_l == e, pi_ref[...], 0), axis=1,
                     keepdims=True)
        sg = seg_ref[...]
        wv = w_ref[...]

        dn = (((1,), (0,)), ((), ()))
        io_n = lax.broadcasted_iota(jnp.int32, (EB, n), 1)
        gi = (mi == io_n).astype(jnp.bfloat16)
        r = lax.dot_general(gi, mt_ref[...], dn,
                            preferred_element_type=jnp.float32)
        io_p = lax.broadcasted_iota(jnp.int32, (EB, p), 1)
        gp = (pi == io_p).astype(jnp.bfloat16)
        r = r + lax.dot_general(gp, pt_ref[...], dn,
                                preferred_element_type=jnp.float32)
        ones_col = (lax.broadcasted_iota(jnp.int32, (EB, 128), 1)
                    == 0).astype(jnp.float32)
        r256 = jnp.concatenate([r, ones_col], axis=1).astype(jnp.bfloat16)

        io_u = lax.broadcasted_iota(jnp.int32, (acc.shape[0], EB), 0)
        sa = jnp.where(sg == io_u, wv, 0.0).astype(jnp.bfloat16)
        acc[...] += lax.dot_general(sa, r256, dn,
                                    preferred_element_type=jnp.float32)

        @pl.when(e == nblk - 1)
        def _():
            den = acc[:, 128:129]
            den = jnp.where(den == 0.0, 1.0, den)
            out_ref[...] = acc[:, :128] / den

    return pl.pallas_call(
        body,
        grid=(nblk,),
        in_specs=[
            pl.BlockSpec((1, EB), lambda e: (0, e)),
            pl.BlockSpec((EB, E // EB), lambda e: (0, 0)),
            pl.BlockSpec((EB, E // EB), lambda e: (0, 0)),
            pl.BlockSpec((1, EB), lambda e: (0, e)),
            pl.BlockSpec((n, d), lambda e: (0, 0)),
            pl.BlockSpec((p, d), lambda e: (0, 0)),
        ],
        out_specs=pl.BlockSpec((n, d), lambda e: (0, 0)),
        out_shape=jax.ShapeDtypeStruct((n, d), jnp.float32),
        scratch_shapes=[pltpu.VMEM((n, 256), jnp.float32)],
    )(seg, msg_idx_t, pos_idx_t, w, msg_tab, pos_tab)


@jax.jit
def kernel(u_emb, i_emb, edge_index, rui, riu, W1, W2, pV, pK):
    U, D = u_emb.shape
    I = i_emb.shape[0]
    E = rui.shape[0]
    P = 256
    u_idx = edge_index[0]
    i_idx = edge_index[1]
    inv_sqrt_d = float(1.0 / np.sqrt(D))

    pvp = jnp.zeros((P, D), jnp.float32).at[:pV.shape[0]].set(pV)
    pkp = jnp.zeros((P, D), jnp.float32).at[:pK.shape[0]].set(pK)

    um, im = _messages_tc(u_emb, i_emb, W1, W2)
    e_tab, a_tab, b_tab = _tables_tc(um, im, pvp, pkp)

    e_flat = e_tab.reshape(U * (I // 128), 128)
    a_flat = a_tab.reshape(U * (P // 128), 128)
    b_flat = b_tab.reshape(I * (P // 128), 128)

    rowe = u_idx * (I // 128) + (i_idx // 128)
    lane_e = i_idx % 128
    rowa = u_idx * (P // 128) + (rui // 128)
    lane_a = rui % 128
    rowb = i_idx * (P // 128) + (riu // 128)
    lane_b = riu % 128

    ea, eb = _edge_exp_sc(e_flat, a_flat, b_flat, rowe, rowa, rowb,
                          lane_e, lane_a, lane_b, inv_sqrt_d)

    seg_u = u_idx.reshape(1, E)
    seg_i = i_idx.reshape(1, E)
    mi_u = i_idx.reshape(E // EB, EB).T
    mi_i = u_idx.reshape(E // EB, EB).T
    pi_u = riu.reshape(E // EB, EB).T
    pi_i = rui.reshape(E // EB, EB).T
    ea2 = ea.reshape(1, E)
    eb2 = eb.reshape(1, E)

    im_bf = im.astype(jnp.bfloat16)
    um_bf = um.astype(jnp.bfloat16)
    pkp_bf = pkp.astype(jnp.bfloat16)
    pvp_bf = pvp.astype(jnp.bfloat16)

    hu = _aggregate_tc(seg_u, mi_u, pi_u, ea2, im_bf, pkp_bf)
    hi = _aggregate_tc(seg_i, mi_i, pi_i, eb2, um_bf, pvp_bf)

    zu = jnp.zeros((U, D), jnp.float32)
    zi = jnp.zeros((I, D), jnp.float32)
    return (hu, zu, hi, zi)
